# Initial kernel scaffold; baseline (speedup 1.0000x reference)
import jax
import jax.numpy as jnp
from jax.experimental import pallas as pl


def kernel(words, table):
    raise NotImplementedError("write your pallas kernel here")


# speedup vs baseline: 1.8806x; 1.8806x over previous
import functools

import jax
import jax.numpy as jnp
from jax import lax
from jax.experimental import pallas as pl
from jax.experimental.pallas import tpu as pltpu
from jax.experimental.pallas import tpu_sc as plsc

_D = 64
_NW = 32
_CHUNK = 128
_NBUF = 4


@functools.partial(jax.jit, static_argnames=("total",))
def _sc_gather(idx3d, table, total):
    b_per_w = total // _NW
    n_chunks = b_per_w // _CHUNK
    assert n_chunks % _NBUF == 0
    mesh = plsc.VectorSubcoreMesh(core_axis_name="c", subcore_axis_name="s")

    @functools.partial(
        pl.kernel,
        out_type=jax.ShapeDtypeStruct((total, _D), jnp.float32),
        mesh=mesh,
        scratch_types=[
            pltpu.VMEM((n_chunks, _CHUNK), jnp.int32),
            pltpu.VMEM((_NBUF, _CHUNK, _D), jnp.float32),
            pltpu.SemaphoreType.DMA((_NBUF,)),
            pltpu.SemaphoreType.DMA((_NBUF,)),
        ],
        compiler_params=pltpu.CompilerParams(use_tc_tiling_on_sc=False),
    )
    def gather_kernel(idx_hbm, table_hbm, out_hbm, idx_v, rows_v, gsem, osem):
        cid = lax.axis_index("c")
        sid = lax.axis_index("s")
        wid = sid * 2 + cid
        base = wid * b_per_w

        pltpu.sync_copy(idx_hbm.at[wid], idx_v)

        def start_gather(jj, b):
            pltpu.async_copy(table_hbm.at[idx_v.at[jj]], rows_v.at[b],
                             gsem.at[b])

        def wait_gather(b):
            pltpu.make_async_copy(table_hbm.at[idx_v.at[0]], rows_v.at[b],
                                  gsem.at[b]).wait()

        def start_out(jj, b):
            pltpu.async_copy(rows_v.at[b],
                             out_hbm.at[pl.ds(base + jj * _CHUNK, _CHUNK)],
                             osem.at[b])

        def wait_out(b):
            pltpu.make_async_copy(rows_v.at[b],
                                  out_hbm.at[pl.ds(base, _CHUNK)],
                                  osem.at[b]).wait()

        for b in range(_NBUF):
            start_gather(b, b)

        @pl.loop(0, n_chunks - _NBUF, step=_NBUF)
        def _body(j):
            for b in range(_NBUF):
                jj = j + b
                wait_gather(b)
                start_out(jj, b)
                wait_out(b)
                start_gather(jj + _NBUF, b)

        for b in range(_NBUF):
            wait_gather(b)
            start_out(n_chunks - _NBUF + b, b)
            wait_out(b)

    return gather_kernel(idx3d, table)


def kernel(words, table):
    batch, hist = words.shape
    total = batch * hist
    idx3d = words.astype(jnp.int32).reshape(
        _NW, total // (_NW * _CHUNK), _CHUNK)
    out = _sc_gather(idx3d, table, total)
    return out.reshape(batch, hist, _D)
